# Initial kernel scaffold; baseline (speedup 1.0000x reference)
import jax
import jax.numpy as jnp
from jax.experimental import pallas as pl


def kernel(x, edge_index, forward_level, forward_index, Ws1, bs1, Ws2, bs2, Ws3, bs3, Wf1, bf1, Wf2, bf2, Wf3, bf3, Wih_s, Whh_s, bih_s, bhh_s, Wih_f, Whh_f, bih_f, bhh_f):
    raise NotImplementedError("write your pallas kernel here")


# speedup vs baseline: 5.3814x; 5.3814x over previous
import functools

import jax
import jax.numpy as jnp
import numpy as np
from jax import lax
from jax.experimental import pallas as pl
from jax.experimental.pallas import tpu as pltpu
from jax.experimental.pallas import tpu_sc as plsc

D = 128
DX = 64
L = 10
BR = 512
NCORE = 2
NSUB = 16
W = NCORE * NSUB
EB = 128
CAP = 8192
NCHUNK = 7
DUMP_DST = 0x7FF00000


def _lane16():
    return lax.iota(jnp.int32, 16)


def _sel(vec16, i):
    return jnp.sum(jnp.where(_lane16() == i, vec16, 0))


def _sc_mesh():
    return plsc.VectorSubcoreMesh(core_axis_name="c", subcore_axis_name="s",
                                  num_cores=NCORE, num_subcores=NSUB)


def _make_sc_gather(n, d, dtype):
    nb_full = n // EB
    tail = n - nb_full * EB
    nbat = nb_full + (1 if tail else 0)
    iters = (nbat + W - 1) // W

    @functools.partial(
        pl.kernel,
        mesh=_sc_mesh(),
        compiler_params=pltpu.CompilerParams(needs_layout_passes=False),
        out_type=jax.ShapeDtypeStruct((n, d), dtype),
        scratch_types=[
            pltpu.VMEM((EB,), jnp.int32),
            pltpu.VMEM((EB, d), dtype),
            pltpu.SemaphoreType.DMA,
        ],
    )
    def k(table_hbm, idx_hbm, out_hbm, idx_v, rows_v, sem):
        w = lax.axis_index("c") * NSUB + lax.axis_index("s")
        for t in range(iters):
            b = w + t * W

            @pl.when(b < nb_full)
            def _():
                e0 = pl.multiple_of(b * EB, EB)
                pltpu.sync_copy(idx_hbm.at[pl.ds(e0, EB)], idx_v)
                pltpu.async_copy(table_hbm.at[idx_v], rows_v, sem).wait()
                pltpu.sync_copy(rows_v, out_hbm.at[pl.ds(e0, EB)])

            if tail:
                @pl.when(b == nb_full)
                def _():
                    e0 = n - EB
                    pltpu.sync_copy(idx_hbm.at[pl.ds(e0, EB)], idx_v)
                    pltpu.async_copy(table_hbm.at[idx_v], rows_v, sem).wait()
                    pltpu.sync_copy(rows_v, out_hbm.at[pl.ds(e0, EB)])

    return k


def _make_bucketize(epw, capb):
    nbat = epw // EB
    nlev = L - 1

    @functools.partial(
        pl.kernel,
        mesh=_sc_mesh(),
        compiler_params=pltpu.CompilerParams(needs_layout_passes=False),
        out_type=[
            jax.ShapeDtypeStruct((nlev * W * capb,), jnp.int32),
            jax.ShapeDtypeStruct((nlev * W * capb,), jnp.int32),
            jax.ShapeDtypeStruct((W * 16,), jnp.int32),
        ],
        scratch_types=[
            pltpu.VMEM((16,), jnp.int32),
            pltpu.VMEM((EB,), jnp.int32),
            pltpu.VMEM((EB,), jnp.int32),
            pltpu.VMEM((16,), jnp.int32),
        ] + [pltpu.VMEM((2 * EB,), jnp.int32) for _ in range(2 * nlev)],
    )
    def k(off_hbm, esrc_hbm, edst_hbm, bsrc_hbm, bdst_hbm, cnt_hbm,
          off_v, sbuf, dbuf, cntv, *stages):
        stsl = stages[:nlev]
        stdl = stages[nlev:]
        w = lax.axis_index("c") * NSUB + lax.axis_index("s")
        pltpu.sync_copy(off_hbm, off_v)
        offv = off_v[...]
        ols = [_sel(offv, i) for i in range(L + 1)]

        def batch_body(b, carry):
            e0 = pl.multiple_of(w * epw + b * EB, EB)
            pltpu.sync_copy(esrc_hbm.at[pl.ds(e0, EB)], sbuf)
            pltpu.sync_copy(edst_hbm.at[pl.ds(e0, EB)], dbuf)
            fills = list(carry[:nlev])
            bases = list(carry[nlev:])
            for i in range(EB // 16):
                dv = dbuf[pl.ds(16 * i, 16)]
                sv = sbuf[pl.ds(16 * i, 16)]
                for l in range(1, L):
                    m = (dv >= ols[l]) & (dv < ols[l + 1])
                    j = l - 1
                    plsc.store_compressed(stdl[j].at[pl.ds(fills[j], 16)],
                                          dv, mask=m)
                    plsc.store_compressed(stsl[j].at[pl.ds(fills[j], 16)],
                                          sv, mask=m)
                    fills[j] = fills[j] + jnp.max(
                        plsc.all_reduce_population_count(m))
            for j in range(nlev):
                def fire(fill, base, j=j):
                    o = pl.multiple_of((j * W + w) * capb + base, EB)
                    pltpu.sync_copy(stsl[j].at[pl.ds(0, EB)],
                                    bsrc_hbm.at[pl.ds(o, EB)])
                    pltpu.sync_copy(stdl[j].at[pl.ds(0, EB)],
                                    bdst_hbm.at[pl.ds(o, EB)])
                    for i in range(EB // 16):
                        vs = stsl[j][pl.ds(EB + 16 * i, 16)]
                        vd = stdl[j][pl.ds(EB + 16 * i, 16)]
                        stsl[j][pl.ds(16 * i, 16)] = vs
                        stdl[j][pl.ds(16 * i, 16)] = vd
                    return fill - EB, base + EB

                fills[j], bases[j] = lax.cond(
                    fills[j] >= EB, fire, lambda f, b_: (f, b_),
                    fills[j], bases[j])
            return tuple(fills) + tuple(bases)

        init = tuple([jnp.int32(0)] * (2 * nlev))
        carry = lax.fori_loop(0, nbat, batch_body, init)
        fills = list(carry[:nlev])
        bases = list(carry[nlev:])

        for j in range(nlev):
            for i in range(2 * EB // 16):
                posm = (16 * i + _lane16()) < fills[j]
                vd = jnp.where(posm, stdl[j][pl.ds(16 * i, 16)],
                               jnp.int32(DUMP_DST))
                vs = jnp.where(posm, stsl[j][pl.ds(16 * i, 16)], jnp.int32(0))
                stdl[j][pl.ds(16 * i, 16)] = vd
                stsl[j][pl.ds(16 * i, 16)] = vs

            def flush(base, j=j, o=0):
                g = pl.multiple_of((j * W + w) * capb + base + o, EB)
                pltpu.sync_copy(stsl[j].at[pl.ds(o, EB)],
                                bsrc_hbm.at[pl.ds(g, EB)])
                pltpu.sync_copy(stdl[j].at[pl.ds(o, EB)],
                                bdst_hbm.at[pl.ds(g, EB)])

            @pl.when(fills[j] > 0)
            def _(j=j):
                flush(bases[j], j=j, o=0)

            @pl.when(fills[j] > EB)
            def _(j=j):
                flush(bases[j], j=j, o=EB)

        total = jnp.zeros((16,), jnp.int32)
        for j in range(nlev):
            total = total + jnp.where(_lane16() == j,
                                      bases[j] + fills[j], 0)
        cntv[...] = total
        pltpu.sync_copy(cntv, cnt_hbm.at[pl.ds(pl.multiple_of(w * 16, 16), 16)])

    return k


def _make_aggr(n, npad, capb):

    @functools.partial(
        pl.kernel,
        mesh=_sc_mesh(),
        compiler_params=pltpu.CompilerParams(needs_layout_passes=False),
        out_type=jax.ShapeDtypeStruct((NCORE, npad, D), jnp.float32),
        scratch_types=[
            pltpu.VMEM((16,), jnp.int32),
            pltpu.VMEM((16,), jnp.int32),
            pltpu.VMEM((EB,), jnp.int32),
            pltpu.VMEM((EB,), jnp.int32),
            pltpu.VMEM((EB,), jnp.int32),
            pltpu.VMEM((EB, D), jnp.float32),
            pltpu.VMEM_SHARED((CAP + 8, D), jnp.float32),
            pltpu.SemaphoreType.DMA,
        ],
    )
    def k(meta_hbm, cnt_hbm, bsrc_hbm, bdst_hbm, table_hbm, zeros_hbm,
          out_hbm, meta_v, cnt_v, sbuf, dbuf, lidx, rows_v, acc, sem):
        c = lax.axis_index("c")
        s = lax.axis_index("s")
        w = c * NSUB + s
        pltpu.sync_copy(meta_hbm, meta_v)
        mv = meta_v[...]
        lo = _sel(mv, 0)
        hi = _sel(mv, 1)
        lm1 = _sel(mv, 2)
        pltpu.sync_copy(cnt_hbm.at[pl.ds(pl.multiple_of(w * 16, 16), 16)],
                        cnt_v)
        cnt = jnp.sum(jnp.where(_lane16() == lm1, cnt_v[...], 0))
        nb = (cnt + EB - 1) // EB

        ZSH = CAP // NSUB

        base0 = pl.multiple_of((lo // 8) * 8, 8)
        for ci in range(NCHUNK):
            rowbase = base0 + ci * CAP
            rowcnt = jnp.clip(hi - rowbase, 0, CAP)

            @pl.when(rowcnt > 0)
            def _(rowbase=rowbase, rowcnt=rowcnt):
                myn = jnp.clip(rowcnt - s * ZSH, 0, ZSH)
                for zi in range(ZSH // EB):
                    @pl.when(myn > zi * EB)
                    def _(zi=zi):
                        pltpu.sync_copy(
                            zeros_hbm, acc.at[pl.ds(s * ZSH + zi * EB, EB)])
                plsc.subcore_barrier()

                def edge_body(b, carry):
                    e0 = pl.multiple_of((lm1 * W + w) * capb + b * EB,
                                        EB)
                    pltpu.sync_copy(bsrc_hbm.at[pl.ds(e0, EB)], sbuf)
                    pltpu.sync_copy(bdst_hbm.at[pl.ds(e0, EB)], dbuf)
                    for i in range(EB // 16):
                        dv = dbuf[pl.ds(16 * i, 16)]
                        lcl = dv - rowbase
                        m = (lcl >= 0) & (lcl < rowcnt)
                        lidx[pl.ds(16 * i, 16)] = jnp.where(
                            m, lcl, jnp.int32(CAP))
                    pltpu.async_copy(table_hbm.at[sbuf], rows_v, sem).wait()
                    pltpu.sync_copy(rows_v, acc.at[lidx], add=True)
                    return carry

                lax.fori_loop(0, nb, edge_body, 0)
                plsc.subcore_barrier()

                for zi in range(ZSH // EB):
                    @pl.when(myn > zi * EB)
                    def _(zi=zi):
                        r0 = s * ZSH + zi * EB
                        g = pl.multiple_of(rowbase + r0, 8)
                        pltpu.sync_copy(
                            acc.at[pl.ds(r0, EB)],
                            out_hbm.at[c, pl.ds(g, EB)])
                plsc.subcore_barrier()

    return k


def _gru(gi, gh, h):
    r = jax.nn.sigmoid(gi[:, :D] + gh[:, :D])
    z = jax.nn.sigmoid(gi[:, D:2 * D] + gh[:, D:2 * D])
    nn = jnp.tanh(gi[:, 2 * D:] + r * gh[:, 2 * D:])
    return (1.0 - z) * nn + z * h


def _mlp3(h0, w1, b1, w2, b2, w3, b3):
    h = jax.nn.relu(jnp.dot(h0, w1, preferred_element_type=jnp.float32) + b1)
    h = jax.nn.relu(jnp.dot(h, w2, preferred_element_type=jnp.float32) + b2)
    return jnp.dot(h, w3, preferred_element_type=jnp.float32) + b3


def _init_body(meta_ref, vec_ref, w1_ref, b1_ref, w2_ref, b2_ref, w3_ref,
               b3_ref, f1_ref, fb1_ref, f2_ref, fb2_ref, f3_ref, fb3_ref,
               hs_ref, ms_ref, mf_ref):
    i = pl.program_id(0)
    count0 = meta_ref[0]
    rows = i * BR + lax.broadcasted_iota(jnp.int32, (BR, 1), 0)
    hs0 = jnp.where(rows < count0, vec_ref[...], 0.0)
    hs_ref[...] = hs0
    ms_ref[...] = _mlp3(hs0, w1_ref[...], b1_ref[...], w2_ref[...],
                        b2_ref[...], w3_ref[...], b3_ref[...])
    h = jax.nn.relu(
        jnp.dot(hs0, f1_ref[...][:D, :], preferred_element_type=jnp.float32)
        + fb1_ref[...])
    h = jax.nn.relu(
        jnp.dot(h, f2_ref[...], preferred_element_type=jnp.float32)
        + fb2_ref[...])
    mf_ref[...] = (jnp.dot(h, f3_ref[...], preferred_element_type=jnp.float32)
                   + fb3_ref[...])


def _stepA_body(meta_ref, msg_ref, xs_ref, hs_ref, wih_ref, bih_ref, whh_ref,
                bhh_ref, w1_ref, b1_ref, w2_ref, b2_ref, w3_ref, b3_ref,
                msprev_ref, hsout_ref, msout_ref):
    i = pl.program_id(0)
    nact = meta_ref[1]

    @pl.when(i < nact)
    def _():
        blk = meta_ref[0] + jnp.minimum(i, nact - 1)
        lo = meta_ref[2]
        hi = meta_ref[3]
        rows = blk * BR + lax.broadcasted_iota(jnp.int32, (BR, 1), 0)
        mask = (rows >= lo) & (rows < hi)
        msg = msg_ref[0] + msg_ref[1]
        hs = hs_ref[...]
        wih = wih_ref[...]
        gi = (jnp.dot(msg, wih[:D, :], preferred_element_type=jnp.float32)
              + jnp.dot(xs_ref[...], wih[D:, :],
                        preferred_element_type=jnp.float32) + bih_ref[...])
        gh = jnp.dot(hs, whh_ref[...],
                     preferred_element_type=jnp.float32) + bhh_ref[...]
        hsnew = jnp.where(mask, _gru(gi, gh, hs), hs)
        hsout_ref[...] = hsnew
        msout_ref[...] = _mlp3(hsnew, w1_ref[...], b1_ref[...], w2_ref[...],
                               b2_ref[...], w3_ref[...], b3_ref[...])


def _stepB_body(meta_ref, msg_ref, xs_ref, hf_ref, hs_ref, wih_ref, bih_ref,
                whh_ref, bhh_ref, f1_ref, fb1_ref, f2_ref, fb2_ref, f3_ref,
                fb3_ref, mfprev_ref, hfout_ref, mfout_ref):
    i = pl.program_id(0)
    nact = meta_ref[1]

    @pl.when(i < nact)
    def _():
        blk = meta_ref[0] + jnp.minimum(i, nact - 1)
        lo = meta_ref[2]
        hi = meta_ref[3]
        rows = blk * BR + lax.broadcasted_iota(jnp.int32, (BR, 1), 0)
        mask = (rows >= lo) & (rows < hi)
        msg = msg_ref[0] + msg_ref[1]
        hf = hf_ref[...]
        wih = wih_ref[...]
        gi = (jnp.dot(msg, wih[:D, :], preferred_element_type=jnp.float32)
              + jnp.dot(xs_ref[...], wih[D:, :],
                        preferred_element_type=jnp.float32) + bih_ref[...])
        gh = jnp.dot(hf, whh_ref[...],
                     preferred_element_type=jnp.float32) + bhh_ref[...]
        hfnew = jnp.where(mask, _gru(gi, gh, hf), hf)
        hfout_ref[...] = hfnew
        f1 = f1_ref[...]
        h = jax.nn.relu(
            jnp.dot(hs_ref[...], f1[:D, :], preferred_element_type=jnp.float32)
            + jnp.dot(hfnew, f1[D:, :], preferred_element_type=jnp.float32)
            + fb1_ref[...])
        h = jax.nn.relu(
            jnp.dot(h, f2_ref[...], preferred_element_type=jnp.float32)
            + fb2_ref[...])
        mfout_ref[...] = (jnp.dot(h, f3_ref[...],
                                  preferred_element_type=jnp.float32)
                          + fb3_ref[...])


def _full(shape):
    ndim = len(shape)
    return pl.BlockSpec(shape, lambda i, m: (0,) * ndim)


def _rowblk(d, npad=None):
    return pl.BlockSpec((BR, d),
                        lambda i, m: (m[0] + jnp.minimum(i, m[1] - 1), 0))


def _msgblk():
    return pl.BlockSpec((NCORE, BR, D),
                        lambda i, m: (0, m[0] + jnp.minimum(i, m[1] - 1), 0))


def kernel(x, edge_index, forward_level, forward_index,
           Ws1, bs1, Ws2, bs2, Ws3, bs3,
           Wf1, bf1, Wf2, bf2, Wf3, bf3,
           Wih_s, Whh_s, bih_s, bhh_s,
           Wih_f, Whh_f, bih_f, bhh_f):
    N = x.shape[0]
    E = edge_index.shape[1]
    NBLK = (N + BR - 1) // BR
    NPAD = N + EB
    EPW = ((E + W * EB - 1) // (W * EB)) * EB
    EPAD = W * EPW
    CAPB = EPW + EB

    u = jax.random.uniform(jax.random.key(7), (N, D), dtype=jnp.float32)
    vec = u - 0.5
    vec = vec / jnp.linalg.norm(vec, axis=1, keepdims=True)

    perm = jnp.argsort(forward_level, stable=True).astype(jnp.int32)
    node_rank = jnp.zeros((N,), jnp.int32).at[perm].set(
        jnp.arange(N, dtype=jnp.int32))
    levels_sorted = forward_level[perm]
    off = jnp.searchsorted(levels_sorted, jnp.arange(L + 1, dtype=jnp.int32),
                           side="left").astype(jnp.int32)
    off_pad = jnp.zeros((16,), jnp.int32).at[:L + 1].set(off)
    esrc = node_rank[edge_index[0]]
    edst = node_rank[edge_index[1]]
    esrc_p = jnp.concatenate(
        [esrc, jnp.zeros((EPAD - E,), jnp.int32)])
    edst_p = jnp.concatenate(
        [edst, jnp.full((EPAD - E,), DUMP_DST, jnp.int32)])

    xs = x[perm]
    bsrc, bdst, counts = _make_bucketize(EPW, CAPB)(off_pad, esrc_p, edst_p)
    aggr = _make_aggr(N, NPAD, CAPB)
    zblk = jnp.zeros((EB, D), jnp.float32)

    weights2d = lambda b: b.reshape(1, -1)
    init_call = pl.pallas_call(
        _init_body,
        grid_spec=pltpu.PrefetchScalarGridSpec(
            num_scalar_prefetch=1,
            grid=(NBLK,),
            in_specs=[
                pl.BlockSpec((BR, D), lambda i, m: (i, 0)),
                _full((D, D)), _full((1, D)), _full((D, D)), _full((1, D)),
                _full((D, D)), _full((1, D)),
                _full((2 * D, D)), _full((1, D)), _full((D, D)), _full((1, D)),
                _full((D, D)), _full((1, D)),
            ],
            out_specs=[
                pl.BlockSpec((BR, D), lambda i, m: (i, 0)),
                pl.BlockSpec((BR, D), lambda i, m: (i, 0)),
                pl.BlockSpec((BR, D), lambda i, m: (i, 0)),
            ],
        ),
        out_shape=[jax.ShapeDtypeStruct((N, D), jnp.float32)] * 3,
    )
    meta0 = jnp.array([0], jnp.int32).at[0].set(off[1])
    hs, MS, MF = init_call(
        meta0, vec, Ws1, weights2d(bs1), Ws2, weights2d(bs2), Ws3,
        weights2d(bs3), Wf1, weights2d(bf1), Wf2, weights2d(bf2), Wf3,
        weights2d(bf3))
    hf = jnp.zeros((N, D), jnp.float32)

    wih_s_t = Wih_s.T
    whh_s_t = Whh_s.T
    wih_f_t = Wih_f.T
    whh_f_t = Whh_f.T

    stepA = pl.pallas_call(
        _stepA_body,
        grid_spec=pltpu.PrefetchScalarGridSpec(
            num_scalar_prefetch=1,
            grid=(NBLK,),
            in_specs=[
                _msgblk(), _rowblk(DX), _rowblk(D),
                _full((D + DX, 3 * D)), _full((1, 3 * D)),
                _full((D, 3 * D)), _full((1, 3 * D)),
                _full((D, D)), _full((1, D)), _full((D, D)), _full((1, D)),
                _full((D, D)), _full((1, D)),
                _rowblk(D),
            ],
            out_specs=[_rowblk(D), _rowblk(D)],
        ),
        out_shape=[jax.ShapeDtypeStruct((N, D), jnp.float32)] * 2,
        input_output_aliases={3: 0, 14: 1},
    )
    stepB = pl.pallas_call(
        _stepB_body,
        grid_spec=pltpu.PrefetchScalarGridSpec(
            num_scalar_prefetch=1,
            grid=(NBLK,),
            in_specs=[
                _msgblk(), _rowblk(DX), _rowblk(D), _rowblk(D),
                _full((D + DX, 3 * D)), _full((1, 3 * D)),
                _full((D, 3 * D)), _full((1, 3 * D)),
                _full((2 * D, D)), _full((1, D)), _full((D, D)),
                _full((1, D)), _full((D, D)), _full((1, D)),
                _rowblk(D),
            ],
            out_specs=[_rowblk(D), _rowblk(D)],
        ),
        out_shape=[jax.ShapeDtypeStruct((N, D), jnp.float32)] * 2,
        input_output_aliases={3: 0, 15: 1},
    )

    for level in range(1, L):
        lo = off[level]
        hi = off[level + 1]
        blo = lo // BR
        nact = jnp.maximum((hi + BR - 1) // BR - blo, 1)
        meta_sc = jnp.zeros((16,), jnp.int32).at[0].set(lo).at[1].set(
            hi).at[2].set(level - 1)
        meta_tc = jnp.stack([blo, nact, lo, hi])

        msg_s = aggr(meta_sc, counts, bsrc, bdst, MS, zblk)
        hs, MS = stepA(meta_tc, msg_s, xs, hs, wih_s_t, weights2d(bih_s),
                       whh_s_t, weights2d(bhh_s), Ws1, weights2d(bs1), Ws2,
                       weights2d(bs2), Ws3, weights2d(bs3), MS)
        msg_f = aggr(meta_sc, counts, bsrc, bdst, MF, zblk)
        hf, MF = stepB(meta_tc, msg_f, xs, hf, hs, wih_f_t, weights2d(bih_f),
                       whh_f_t, weights2d(bhh_f), Wf1, weights2d(bf1), Wf2,
                       weights2d(bf2), Wf3, weights2d(bf3), MF)

    out = _make_sc_gather(N, D, jnp.float32)(hf, node_rank)
    return out
